# baseline (speedup 1.0000x reference)
import functools

import jax
import jax.numpy as jnp
from jax import lax
from jax.experimental import pallas as pl
from jax.experimental.pallas import tpu as pltpu
from jax.experimental.pallas import tpu_sc as plsc

VOCAB = 49408
CTX_DIM = 512
PROJ_DIM = 768
N_CTX = 2
B = 1024
SEQ = 77
FLAT = B * SEQ

_info = plsc.get_sparse_core_info()
NC, NS = _info.num_cores, _info.num_subcores
NW = NC * NS
PER_W = FLAT // NW
G = 112
NCH = PER_W // G

_W0_CTX = (
    [(B, 96, 0)]
    + [(1120 + k * G, G, 0) for k in range(8)]
    + [(2016, 32, 0)]
    + [(2 * B + k * G, G, 1) for k in range(3)]
    + [(2384, 80, 1)]
)
_W1_CTX = (
    [(PER_W + k * G, G, 1) for k in range(5)]
    + [(3024, 48, 1)]
)


def _sc_gather(idx_flat, table, ctx, fill01):
    mesh = plsc.VectorSubcoreMesh(core_axis_name="c", subcore_axis_name="s")

    @functools.partial(
        pl.kernel,
        mesh=mesh,
        out_type=jax.ShapeDtypeStruct((FLAT, CTX_DIM), jnp.float32),
        scratch_types=[
            pltpu.VMEM((PER_W,), jnp.int32),
            pltpu.VMEM((G, CTX_DIM), jnp.float32),
            pltpu.VMEM((G, CTX_DIM), jnp.float32),
            pltpu.VMEM((N_CTX, G), jnp.int32),
            pltpu.SemaphoreType.DMA,
            pltpu.SemaphoreType.DMA,
            pltpu.SemaphoreType.DMA,
            pltpu.SemaphoreType.DMA,
            pltpu.SemaphoreType.DMA,
        ],
    )
    def body(idx_hbm, table_hbm, ctx_hbm, fill_hbm, out_hbm,
             idx_v, rows_v0, rows_v1, fill_v,
             sem_g0, sem_g1, sem_o0, sem_o1, sem_c):
        wid = lax.axis_index("s") * NC + lax.axis_index("c")
        span = jnp.where(wid == 1, 31, jnp.where(wid == 31, 1, wid))
        base = span * PER_W

        rows_v = (rows_v0, rows_v1)
        sem_g = (sem_g0, sem_g1)
        sem_o = (sem_o0, sem_o1)

        pltpu.sync_copy(idx_hbm.at[pl.ds(base, PER_W)], idx_v)

        def ring(start, n):
            gd = [None, None]
            od = [None, None]

            def off(j):
                return base + j * G

            gd[0] = pltpu.async_copy(
                table_hbm.at[idx_v.at[pl.ds(start * G, G)]], rows_v0, sem_g0)
            for t in range(1, n):
                j = start + t
                p, q = t & 1, (t - 1) & 1
                gd[q].wait()
                od[q] = pltpu.async_copy(
                    rows_v[q], out_hbm.at[pl.ds(off(j - 1), G)], sem_o[q])
                if od[p] is not None:
                    od[p].wait()
                gd[p] = pltpu.async_copy(
                    table_hbm.at[idx_v.at[pl.ds(j * G, G)]], rows_v[p], sem_g[p])
            last = (n - 1) & 1
            gd[last].wait()
            od[last] = pltpu.async_copy(
                rows_v[last], out_hbm.at[pl.ds(off(start + n - 1), G)],
                sem_o[last])
            if od[1 - last] is not None:
                od[1 - last].wait()
            od[last].wait()

        def ctx_writes(copies, need_ctx0):
            pltpu.sync_copy(fill_hbm, fill_v)
            if need_ctx0:
                pltpu.async_copy(
                    ctx_hbm.at[fill_v.at[0]], rows_v0, sem_c).wait()
            pltpu.async_copy(ctx_hbm.at[fill_v.at[1]], rows_v1, sem_c).wait()
            ds = []
            for o, n, which in copies:
                src = rows_v1 if which else rows_v0
                ds.append(pltpu.async_copy(
                    src.at[pl.ds(0, n)], out_hbm.at[pl.ds(o, n)], sem_c))
            for d in ds:
                d.wait()

        @pl.when(wid == 0)
        def _():
            ring(0, 10)
            ctx_writes(_W0_CTX, True)

        @pl.when(wid == 31)
        def _():
            ring(5, NCH - 5)
            ctx_writes(_W1_CTX, False)

        @pl.when(jnp.logical_and(wid >= 1, wid <= 30))
        def _():
            ring(0, NCH)

    return body(idx_flat, table, ctx, fill01)


def _tc_matmuls(ctx8, proj_W, proj_b2, cpt8, cW, cb2):
    def body(a_ref, w1_ref, b1_ref, c_ref, w2_ref, b2_ref, o1_ref, o2_ref):
        o1_ref[...] = (
            lax.dot_general(
                a_ref[...], w1_ref[...], (((1,), (1,)), ((), ())),
                preferred_element_type=jnp.float32,
            )
            + b1_ref[...]
        )
        o2_ref[...] = (
            lax.dot_general(
                c_ref[...], w2_ref[...], (((1,), (1,)), ((), ())),
                preferred_element_type=jnp.float32,
            )
            + b2_ref[...]
        )

    o1, o2 = pl.pallas_call(
        body,
        out_shape=(
            jax.ShapeDtypeStruct((8, PROJ_DIM), jnp.float32),
            jax.ShapeDtypeStruct((8, PROJ_DIM), jnp.float32),
        ),
    )(ctx8, proj_W, proj_b2, cpt8, cW, cb2)
    return o1, o2


def kernel(text, token_embedding, ctx, proj_W, proj_b, cpt0, cW, cb):
    t = text.astype(jnp.int32)
    zeros = jnp.zeros((B, N_CTX), jnp.int32)
    pt_int = jnp.concatenate(
        [t[:, 0:1], zeros, t[:, 1 : SEQ - 1 - N_CTX], t[:, SEQ - 1 :]], axis=1
    )
    prompt_token = pt_int.astype(jnp.float32)

    idx_flat = pt_int.T.reshape(-1)
    fill01 = jnp.concatenate(
        [jnp.zeros((1, G), jnp.int32), jnp.ones((1, G), jnp.int32)], axis=0
    )

    out_flat = _sc_gather(idx_flat, token_embedding, ctx, fill01)
    prompt_embedding = out_flat.reshape(SEQ, B, CTX_DIM).transpose(1, 0, 2)

    ctx8 = jnp.pad(ctx, ((0, 8 - N_CTX), (0, 0)))
    cpt8 = jnp.pad(cpt0, ((0, 8 - N_CTX), (0, 0)))
    o1, o2 = _tc_matmuls(ctx8, proj_W, proj_b[None, :], cpt8, cW, cb[None, :])
    proj_ctx = o1[:N_CTX]
    visual0 = o2[:N_CTX]

    return (prompt_embedding, prompt_token, proj_ctx, cpt0, visual0)


# speedup vs baseline: 1.2818x; 1.2818x over previous
import functools

import jax
import jax.numpy as jnp
from jax import lax
from jax.experimental import pallas as pl
from jax.experimental.pallas import tpu as pltpu
from jax.experimental.pallas import tpu_sc as plsc

VOCAB = 49408
CTX_DIM = 512
PROJ_DIM = 768
N_CTX = 2
B = 1024
SEQ = 77
FLAT = B * SEQ

_info = plsc.get_sparse_core_info()
NC, NS = _info.num_cores, _info.num_subcores
NW = NC * NS
PER_W = FLAT // NW
G = 112
NCH = PER_W // G

_W0_CTX = (
    [(B, 96, 0)]
    + [(1120 + k * G, G, 0) for k in range(8)]
    + [(2016, 32, 0)]
    + [(2 * B + k * G, G, 1) for k in range(3)]
    + [(2384, 80, 1)]
)
_W1_CTX = (
    [(PER_W + k * G, G, 1) for k in range(5)]
    + [(3024, 48, 1)]
)


def _sc_gather(idx_flat, table, ctx, fill01):
    mesh = plsc.VectorSubcoreMesh(core_axis_name="c", subcore_axis_name="s")

    @functools.partial(
        pl.kernel,
        mesh=mesh,
        out_type=jax.ShapeDtypeStruct((FLAT, CTX_DIM), jnp.float32),
        scratch_types=[
            pltpu.VMEM((PER_W,), jnp.int32),
            pltpu.VMEM((G, CTX_DIM), jnp.float32),
            pltpu.VMEM((G, CTX_DIM), jnp.float32),
            pltpu.VMEM((N_CTX, G), jnp.int32),
            pltpu.SemaphoreType.DMA,
            pltpu.SemaphoreType.DMA,
            pltpu.SemaphoreType.DMA,
            pltpu.SemaphoreType.DMA,
            pltpu.SemaphoreType.DMA,
        ],
    )
    def body(idx_hbm, table_hbm, ctx_hbm, fill_hbm, out_hbm,
             idx_v, rows_v0, rows_v1, fill_v,
             sem_g0, sem_g1, sem_o0, sem_o1, sem_c):
        wid = lax.axis_index("s") * NC + lax.axis_index("c")
        base = wid * PER_W

        rows_v = (rows_v0, rows_v1)
        sem_g = (sem_g0, sem_g1)
        sem_o = (sem_o0, sem_o1)

        pltpu.sync_copy(idx_hbm.at[pl.ds(base, PER_W)], idx_v)

        def ring(start, n):
            gd = [None, None]
            od = [None, None]

            def off(j):
                return base + j * G

            gd[0] = pltpu.async_copy(
                table_hbm.at[idx_v.at[pl.ds(start * G, G)]], rows_v0, sem_g0)
            for t in range(1, n):
                j = start + t
                p, q = t & 1, (t - 1) & 1
                gd[q].wait()
                od[q] = pltpu.async_copy(
                    rows_v[q], out_hbm.at[pl.ds(off(j - 1), G)], sem_o[q])
                if od[p] is not None:
                    od[p].wait()
                gd[p] = pltpu.async_copy(
                    table_hbm.at[idx_v.at[pl.ds(j * G, G)]], rows_v[p], sem_g[p])
            last = (n - 1) & 1
            gd[last].wait()
            od[last] = pltpu.async_copy(
                rows_v[last], out_hbm.at[pl.ds(off(start + n - 1), G)],
                sem_o[last])
            if od[1 - last] is not None:
                od[1 - last].wait()
            od[last].wait()

        def ctx_writes(copies, need_ctx0):
            pltpu.sync_copy(fill_hbm, fill_v)
            if need_ctx0:
                pltpu.async_copy(
                    ctx_hbm.at[fill_v.at[0]], rows_v0, sem_c).wait()
            pltpu.async_copy(ctx_hbm.at[fill_v.at[1]], rows_v1, sem_c).wait()
            ds = []
            for o, n, which in copies:
                src = rows_v1 if which else rows_v0
                ds.append(pltpu.async_copy(
                    src.at[pl.ds(0, n)], out_hbm.at[pl.ds(o, n)], sem_c))
            for d in ds:
                d.wait()

        @pl.when(wid == 0)
        def _():
            ring(0, 10)
            ctx_writes(_W0_CTX, True)

        @pl.when(wid == 1)
        def _():
            ring(5, NCH - 5)
            ctx_writes(_W1_CTX, False)

        @pl.when(wid >= 2)
        def _():
            ring(0, NCH)

    return body(idx_flat, table, ctx, fill01)


def _tc_matmuls(ctx8, proj_W, proj_b2, cpt8, cW, cb2):
    def body(a_ref, w1_ref, b1_ref, c_ref, w2_ref, b2_ref, o1_ref, o2_ref):
        o1_ref[...] = (
            lax.dot_general(
                a_ref[...], w1_ref[...], (((1,), (1,)), ((), ())),
                preferred_element_type=jnp.float32,
            )
            + b1_ref[...]
        )
        o2_ref[...] = (
            lax.dot_general(
                c_ref[...], w2_ref[...], (((1,), (1,)), ((), ())),
                preferred_element_type=jnp.float32,
            )
            + b2_ref[...]
        )

    o1, o2 = pl.pallas_call(
        body,
        out_shape=(
            jax.ShapeDtypeStruct((8, PROJ_DIM), jnp.float32),
            jax.ShapeDtypeStruct((8, PROJ_DIM), jnp.float32),
        ),
    )(ctx8, proj_W, proj_b2, cpt8, cW, cb2)
    return o1, o2


def kernel(text, token_embedding, ctx, proj_W, proj_b, cpt0, cW, cb):
    t = text.astype(jnp.int32)
    zeros = jnp.zeros((B, N_CTX), jnp.int32)
    pt_int = jnp.concatenate(
        [t[:, 0:1], zeros, t[:, 1 : SEQ - 1 - N_CTX], t[:, SEQ - 1 :]], axis=1
    )
    prompt_token = pt_int.astype(jnp.float32)

    idx_flat = pt_int.T.reshape(-1)
    fill01 = jnp.concatenate(
        [jnp.zeros((1, G), jnp.int32), jnp.ones((1, G), jnp.int32)], axis=0
    )

    out_flat = _sc_gather(idx_flat, token_embedding, ctx, fill01)
    prompt_embedding = out_flat.reshape(SEQ, B, CTX_DIM).transpose(1, 0, 2)

    ctx8 = jnp.pad(ctx, ((0, 8 - N_CTX), (0, 0)))
    cpt8 = jnp.pad(cpt0, ((0, 8 - N_CTX), (0, 0)))
    o1, o2 = _tc_matmuls(ctx8, proj_W, proj_b[None, :], cpt8, cW, cb[None, :])
    proj_ctx = o1[:N_CTX]
    visual0 = o2[:N_CTX]

    return (prompt_embedding, prompt_token, proj_ctx, cpt0, visual0)
